# Initial kernel scaffold; baseline (speedup 1.0000x reference)
import jax
import jax.numpy as jnp
from jax.experimental import pallas as pl


def kernel(position_feature, id_feature, edge_index, temporal_edge_index, batch, pos_W, pos_b, id_emb, node_W, node_b, conv_W, conv_b, agg_W, agg_b):
    raise NotImplementedError("write your pallas kernel here")


# speedup vs baseline: 8.3738x; 8.3738x over previous
import functools

import jax
import jax.numpy as jnp
from jax import lax
from jax.experimental import pallas as pl
from jax.experimental.pallas import tpu as pltpu
from jax.experimental.pallas import tpu_sc as plsc

N = 10000
NP = 10240
E = 160000
EP = 163840
D = 256
H = 128
NG = 64
NV = 300
CHUNK = 128
CPT = EP // 16 // CHUNK
DW = 16
ECH = 64
NCNK = EP // 16 // ECH
PAIRS = NCNK // 2
RB = 1280
GRID = NP // RB
NPT = NP // 16


def _deg_body(dst_hbm, deg_hbm, idx_v, ones_v, zbuf_v, shared):
    c = lax.axis_index("c")
    t = lax.axis_index("s")

    def _init(i, _):
        ones_v[i, :] = jnp.ones((DW,), jnp.float32)
        zbuf_v[i, :] = jnp.zeros((DW,), jnp.float32)
        return 0

    lax.fori_loop(0, CHUNK, _init, 0)
    for kk in range(NPT // CHUNK):
        pltpu.sync_copy(zbuf_v, shared.at[pl.ds(t * NPT + kk * CHUNK, CHUNK)])
    plsc.subcore_barrier()

    def _step(k, _):
        pltpu.sync_copy(dst_hbm.at[c, t, k], idx_v)
        pltpu.sync_copy(ones_v, shared.at[idx_v], add=True)
        return 0

    lax.fori_loop(0, CPT, _step, 0)
    plsc.subcore_barrier()
    pltpu.sync_copy(shared.at[pl.ds(t * NPT, NPT)],
                    deg_hbm.at[c, pl.ds(t * NPT, NPT)])


def _conv_body(z_hbm, srcs_hbm, dst_hbm, out_hbm,
               is0, is1, id0, id1, rows0, rows1, shared, sg0, sg1):
    c = lax.axis_index("c")
    t = lax.axis_index("s")
    base = t * NPT

    def _zb(i, _):
        for j in range(H // 16):
            rows0[i, pl.ds(j * 16, 16)] = jnp.zeros((16,), jnp.float32)
        return 0

    lax.fori_loop(0, ECH, _zb, 0)
    for kk in range(NPT // ECH):
        pltpu.sync_copy(rows0, shared.at[pl.ds(base + kk * ECH, ECH)])
    plsc.subcore_barrier()

    pltpu.sync_copy(srcs_hbm.at[c, t, 0], is0)
    pltpu.async_copy(z_hbm.at[is0], rows0, sg0)
    pltpu.sync_copy(srcs_hbm.at[c, t, 1], is1)
    pltpu.async_copy(z_hbm.at[is1], rows1, sg1)

    def _step(k, _):
        pltpu.make_async_copy(z_hbm.at[is0], rows0, sg0).wait()
        pltpu.sync_copy(dst_hbm.at[t, 2 * k], id0)
        pltpu.sync_copy(rows0, shared.at[id0], add=True)
        pltpu.sync_copy(srcs_hbm.at[c, t, 2 * k + 2], is0)
        pltpu.async_copy(z_hbm.at[is0], rows0, sg0)
        pltpu.make_async_copy(z_hbm.at[is1], rows1, sg1).wait()
        pltpu.sync_copy(dst_hbm.at[t, 2 * k + 1], id1)
        pltpu.sync_copy(rows1, shared.at[id1], add=True)
        pltpu.sync_copy(srcs_hbm.at[c, t, 2 * k + 3], is1)
        pltpu.async_copy(z_hbm.at[is1], rows1, sg1)
        return 0

    lax.fori_loop(0, PAIRS, _step, 0)
    pltpu.make_async_copy(z_hbm.at[is0], rows0, sg0).wait()
    pltpu.make_async_copy(z_hbm.at[is1], rows1, sg1).wait()
    plsc.subcore_barrier()
    pltpu.sync_copy(shared.at[pl.ds(base, NPT)],
                    out_hbm.at[c, pl.ds(base, NPT)])


@functools.cache
def _sc_kernels():
    mesh = plsc.VectorSubcoreMesh(core_axis_name="c", subcore_axis_name="s",
                                  num_cores=2, num_subcores=16)
    deg = pl.kernel(
        _deg_body,
        out_type=jax.ShapeDtypeStruct((2, NP, DW), jnp.float32),
        mesh=mesh,
        scratch_types=[
            pltpu.VMEM((CHUNK,), jnp.int32),
            pltpu.VMEM((CHUNK, DW), jnp.float32),
            pltpu.VMEM((CHUNK, DW), jnp.float32),
            pltpu.VMEM_SHARED((NP, DW), jnp.float32),
        ],
    )
    conv = pl.kernel(
        _conv_body,
        out_type=jax.ShapeDtypeStruct((2, NP, H), jnp.float32),
        mesh=mesh,
        scratch_types=[
            pltpu.VMEM((ECH,), jnp.int32),
            pltpu.VMEM((ECH,), jnp.int32),
            pltpu.VMEM((ECH,), jnp.int32),
            pltpu.VMEM((ECH,), jnp.int32),
            pltpu.VMEM((ECH, H), jnp.float32),
            pltpu.VMEM((ECH, H), jnp.float32),
            pltpu.VMEM_SHARED((NP, H), jnp.float32),
            pltpu.SemaphoreType.DMA,
            pltpu.SemaphoreType.DMA,
        ],
    )
    return deg, conv


def _k0_body(p_ref, id_ref, dege_ref, posW_ref, posb_ref, emb_ref,
             wt_ref, wb_ref, nodeb_ref, w0_ref, n0_ref, z0_ref):
    pos = jnp.maximum(
        jnp.dot(p_ref[...], posW_ref[...], preferred_element_type=jnp.float32)
        + posb_ref[...], 0.0)
    iota = lax.broadcasted_iota(jnp.int32, (RB, NV), 1)
    oh = (iota == id_ref[...]).astype(jnp.float32)
    idf = jnp.dot(oh, jnp.maximum(emb_ref[...], 0.0),
                  preferred_element_type=jnp.float32)
    n0 = jnp.maximum(
        jnp.dot(pos, wt_ref[...], preferred_element_type=jnp.float32)
        + jnp.dot(idf, wb_ref[...], preferred_element_type=jnp.float32)
        + nodeb_ref[...], 0.0)
    dis = lax.rsqrt(dege_ref[...] + 1.0)
    z0 = jnp.dot(n0, w0_ref[...], preferred_element_type=jnp.float32) * dis
    n0_ref[...] = n0
    z0_ref[0] = z0[:, :H]
    z0_ref[1] = z0[:, H:]


def _full(shape):
    return pl.BlockSpec(shape, lambda i: tuple(0 for _ in shape))


def _k0(p, ids, deg_e, pos_W, pos_b, emb, node_W, node_b, w0):
    return pl.pallas_call(
        _k0_body,
        grid=(GRID,),
        in_specs=[
            pl.BlockSpec((RB, 3), lambda i: (i, 0)),
            pl.BlockSpec((RB, 1), lambda i: (i, 0)),
            pl.BlockSpec((RB, 1), lambda i: (i, 0)),
            _full((3, D)),
            _full((1, D)),
            _full((NV, D)),
            _full((D, D)),
            _full((D, D)),
            _full((1, D)),
            _full((D, D)),
        ],
        out_specs=[
            pl.BlockSpec((RB, D), lambda i: (i, 0)),
            pl.BlockSpec((2, RB, H), lambda i: (0, i, 0)),
        ],
        out_shape=[
            jax.ShapeDtypeStruct((NP, D), jnp.float32),
            jax.ShapeDtypeStruct((2, NP, H), jnp.float32),
        ],
    )(p, ids, deg_e, pos_W, pos_b.reshape(1, D), emb,
      node_W[:D], node_W[D:], node_b.reshape(1, D), w0)


def _stage_body(out_x, s_ref, z_ref, degc_ref, degn_ref, b_ref,
                w_ref, *out_refs):
    s = jnp.concatenate([s_ref[0], s_ref[1]], axis=1)
    z = jnp.concatenate([z_ref[0], z_ref[1]], axis=1)
    dis = lax.rsqrt(degc_ref[...] + 1.0)
    x = jnp.maximum(dis * (s + z) + b_ref[...], 0.0)
    zn = jnp.dot(x, w_ref[...], preferred_element_type=jnp.float32) \
        * lax.rsqrt(degn_ref[...] + 1.0)
    out_refs[0][0] = zn[:, :H]
    out_refs[0][1] = zn[:, H:]
    if out_x:
        out_refs[1][...] = x


def _stage(s2, z2, deg_c, deg_n, b, w, out_x):
    out_specs = [pl.BlockSpec((2, RB, H), lambda i: (0, i, 0))]
    out_shape = [jax.ShapeDtypeStruct((2, NP, H), jnp.float32)]
    if out_x:
        out_specs.append(pl.BlockSpec((RB, D), lambda i: (i, 0)))
        out_shape.append(jax.ShapeDtypeStruct((NP, D), jnp.float32))
    return pl.pallas_call(
        functools.partial(_stage_body, out_x),
        grid=(GRID,),
        in_specs=[
            pl.BlockSpec((2, RB, H), lambda i: (0, i, 0)),
            pl.BlockSpec((2, RB, H), lambda i: (0, i, 0)),
            pl.BlockSpec((RB, 1), lambda i: (i, 0)),
            pl.BlockSpec((RB, 1), lambda i: (i, 0)),
            _full((1, D)),
            _full((D, D)),
        ],
        out_specs=out_specs,
        out_shape=out_shape,
    )(s2, z2, deg_c, deg_n, b.reshape(1, D), w)


def _last_body(s_ref, z_ref, degc_ref, b_ref, x_ref):
    s = jnp.concatenate([s_ref[0], s_ref[1]], axis=1)
    z = jnp.concatenate([z_ref[0], z_ref[1]], axis=1)
    dis = lax.rsqrt(degc_ref[...] + 1.0)
    x_ref[...] = jnp.maximum(dis * (s + z) + b_ref[...], 0.0)


def _last(s2, z2, deg_c, b):
    return pl.pallas_call(
        _last_body,
        grid=(GRID,),
        in_specs=[
            pl.BlockSpec((2, RB, H), lambda i: (0, i, 0)),
            pl.BlockSpec((2, RB, H), lambda i: (0, i, 0)),
            pl.BlockSpec((RB, 1), lambda i: (i, 0)),
            _full((1, D)),
        ],
        out_specs=pl.BlockSpec((RB, D), lambda i: (i, 0)),
        out_shape=jax.ShapeDtypeStruct((NP, D), jnp.float32),
    )(s2, z2, deg_c, b.reshape(1, D))


def _pool_body(batch_ref, a_ref, b_ref, c_ref, g_ref):
    g = pl.program_id(0)
    bvals = batch_ref[...]
    start = jnp.sum((bvals < g).astype(jnp.int32))
    cnt = jnp.sum((bvals == g).astype(jnp.int32))
    c0 = start // 8
    c1 = (start + cnt + 7) // 8
    neg = jnp.full((8, D), -jnp.inf, jnp.float32)

    def _cond(carry):
        return carry[0] < c1

    def _body(carry):
        c, m0, m1, m2 = carry
        r = pl.ds(c * 8, 8)
        bm = batch_ref[r, :] == g

        def mx(acc, ref):
            return jnp.maximum(acc, jnp.where(bm, ref[r, :], -jnp.inf))

        return (c + 1, mx(m0, a_ref), mx(m1, b_ref), mx(m2, c_ref))

    _, m0, m1, m2 = lax.while_loop(_cond, _body, (c0, neg, neg, neg))
    g_ref[0, :, 0:D] = jnp.max(m0, axis=0, keepdims=True)
    g_ref[0, :, D:2 * D] = jnp.max(m1, axis=0, keepdims=True)
    g_ref[0, :, 2 * D:3 * D] = jnp.max(m2, axis=0, keepdims=True)


def _pool(batch_col, a, b, c):
    return pl.pallas_call(
        _pool_body,
        grid=(NG,),
        in_specs=[
            _full((NP, 1)),
            _full((NP, D)),
            _full((NP, D)),
            _full((NP, D)),
        ],
        out_specs=pl.BlockSpec((1, 1, 3 * D), lambda g: (g, 0, 0)),
        out_shape=jax.ShapeDtypeStruct((NG, 1, 3 * D), jnp.float32),
    )(batch_col, a, b, c).reshape(NG, 3 * D)


def _final_body(g_ref, w_ref, b_ref, out_ref):
    out_ref[...] = jnp.dot(g_ref[...], w_ref[...],
                           preferred_element_type=jnp.float32) + b_ref[...]


def _final(g, w, b):
    return pl.pallas_call(
        _final_body,
        grid=(1,),
        in_specs=[_full((NG, 3 * D)), _full((3 * D, D)), _full((1, D))],
        out_specs=_full((NG, D)),
        out_shape=jax.ShapeDtypeStruct((NG, D), jnp.float32),
    )(g, w, b.reshape(1, D))


def kernel(position_feature, id_feature, edge_index, temporal_edge_index,
           batch, pos_W, pos_b, id_emb, node_W, node_b, conv_W, conv_b,
           agg_W, agg_b):
    f32 = jnp.float32
    i32 = jnp.int32

    p = jnp.zeros((NP, 3), f32).at[:N].set(position_feature.astype(f32))
    ids = jnp.zeros((NP, 1), i32).at[:N].set(id_feature.astype(i32))
    batch_col = jnp.full((NP, 1), NG, i32).at[:N, 0].set(batch.astype(i32))

    pad_i = N + (jnp.arange(EP - E, dtype=i32) % (NP - N))
    dummy = jnp.broadcast_to(
        (N + (jnp.arange(2 * ECH, dtype=i32) % (NP - N))).reshape(1, 2, ECH),
        (16, 2, ECH))

    def prep(ei):
        s = jnp.concatenate([ei[0].astype(i32), pad_i]).reshape(16, NCNK, ECH)
        d = jnp.concatenate([ei[1].astype(i32), pad_i]).reshape(16, NCNK, ECH)
        s = jnp.concatenate([s, dummy], axis=1)
        srcs = jnp.stack([s, s + NP])
        return srcs, d

    srcs_e, dst_e = prep(edge_index)
    srcs_t, dst_t = prep(temporal_edge_index)

    _deg_kernel, _conv_kernel = _sc_kernels()
    counts = _deg_kernel(
        jnp.stack([dst_e, dst_t]).reshape(2, 16, CPT, CHUNK))
    deg_e = counts[0, :, :1]
    deg_t = counts[1, :, :1]

    n0, z0 = _k0(p, ids, deg_e, pos_W, pos_b, id_emb, node_W, node_b,
                 conv_W[0])
    s0 = _conv_kernel(z0.reshape(2 * NP, H), srcs_e, dst_e)
    (z1,) = _stage(s0, z0, deg_e, deg_t, conv_b[0], conv_W[1], out_x=False)
    s1 = _conv_kernel(z1.reshape(2 * NP, H), srcs_t, dst_t)
    z2, x2 = _stage(s1, z1, deg_t, deg_e, conv_b[1], conv_W[2], out_x=True)
    s2 = _conv_kernel(z2.reshape(2 * NP, H), srcs_e, dst_e)
    (z3,) = _stage(s2, z2, deg_e, deg_t, conv_b[2], conv_W[3], out_x=False)
    s3 = _conv_kernel(z3.reshape(2 * NP, H), srcs_t, dst_t)
    x4 = _last(s3, z3, deg_t, conv_b[3])

    g = _pool(batch_col, n0, x2, x4)
    return _final(g, agg_W, agg_b)


# speedup vs baseline: 11.3083x; 1.3504x over previous
import functools

import jax
import jax.numpy as jnp
from jax import lax
from jax.experimental import pallas as pl
from jax.experimental.pallas import tpu as pltpu
from jax.experimental.pallas import tpu_sc as plsc

N = 10000
NP = 10240
E = 160000
EP = 163840
D = 256
H = 128
NG = 64
NV = 300
CHUNK = 128
CPT = EP // 16 // CHUNK
DW = 16
ECH = 64
NCNK = EP // 16 // ECH
PAIRS = NCNK // 2
RB = 1280
GRID = NP // RB
NPT = NP // 16


def _deg_body(dst_hbm, deg_hbm, idx_v, ones_v, zbuf_v, shared):
    c = lax.axis_index("c")
    t = lax.axis_index("s")

    def _init(i, _):
        ones_v[i, :] = jnp.ones((DW,), jnp.float32)
        zbuf_v[i, :] = jnp.zeros((DW,), jnp.float32)
        return 0

    lax.fori_loop(0, CHUNK, _init, 0)
    for kk in range(NPT // CHUNK):
        pltpu.sync_copy(zbuf_v, shared.at[pl.ds(t * NPT + kk * CHUNK, CHUNK)])
    plsc.subcore_barrier()

    def _step(k, _):
        pltpu.sync_copy(dst_hbm.at[c, t, k], idx_v)
        pltpu.sync_copy(ones_v, shared.at[idx_v], add=True)
        return 0

    lax.fori_loop(0, CPT, _step, 0)
    plsc.subcore_barrier()
    pltpu.sync_copy(shared.at[pl.ds(t * NPT, NPT)],
                    deg_hbm.at[c, pl.ds(t * NPT, NPT)])


def _conv_body(z_hbm, srcs_hbm, dst_hbm, out_hbm,
               is0, is1, dslab, rows0, rows1, shared, sg0, sg1, si0, si1):
    c = lax.axis_index("c")
    t = lax.axis_index("s")
    base = t * NPT

    def _zb(i, _):
        for j in range(H // 16):
            rows0[i, pl.ds(j * 16, 16)] = jnp.zeros((16,), jnp.float32)
        return 0

    lax.fori_loop(0, ECH, _zb, 0)
    for kk in range(NPT // ECH):
        pltpu.sync_copy(rows0, shared.at[pl.ds(base + kk * ECH, ECH)])
    pltpu.sync_copy(dst_hbm.at[t], dslab)
    plsc.subcore_barrier()

    pltpu.sync_copy(srcs_hbm.at[c, t, 0], is0)
    pltpu.async_copy(z_hbm.at[is0], rows0, sg0)
    pltpu.sync_copy(srcs_hbm.at[c, t, 1], is1)
    pltpu.async_copy(z_hbm.at[is1], rows1, sg1)

    def _step(k, _):
        pltpu.make_async_copy(z_hbm.at[is0], rows0, sg0).wait()
        pltpu.async_copy(srcs_hbm.at[c, t, 2 * k + 2], is0, si0)
        pltpu.sync_copy(rows0, shared.at[dslab.at[2 * k]], add=True)
        pltpu.make_async_copy(srcs_hbm.at[c, t, 2 * k + 2], is0, si0).wait()
        pltpu.async_copy(z_hbm.at[is0], rows0, sg0)
        pltpu.make_async_copy(z_hbm.at[is1], rows1, sg1).wait()
        pltpu.async_copy(srcs_hbm.at[c, t, 2 * k + 3], is1, si1)
        pltpu.sync_copy(rows1, shared.at[dslab.at[2 * k + 1]], add=True)
        pltpu.make_async_copy(srcs_hbm.at[c, t, 2 * k + 3], is1, si1).wait()
        pltpu.async_copy(z_hbm.at[is1], rows1, sg1)
        return 0

    lax.fori_loop(0, PAIRS, _step, 0)
    pltpu.make_async_copy(z_hbm.at[is0], rows0, sg0).wait()
    pltpu.make_async_copy(z_hbm.at[is1], rows1, sg1).wait()
    plsc.subcore_barrier()
    pltpu.sync_copy(shared.at[pl.ds(base, NPT)],
                    out_hbm.at[c, pl.ds(base, NPT)])


@functools.cache
def _sc_kernels():
    mesh = plsc.VectorSubcoreMesh(core_axis_name="c", subcore_axis_name="s",
                                  num_cores=2, num_subcores=16)
    deg = pl.kernel(
        _deg_body,
        out_type=jax.ShapeDtypeStruct((2, NP, DW), jnp.float32),
        mesh=mesh,
        scratch_types=[
            pltpu.VMEM((CHUNK,), jnp.int32),
            pltpu.VMEM((CHUNK, DW), jnp.float32),
            pltpu.VMEM((CHUNK, DW), jnp.float32),
            pltpu.VMEM_SHARED((NP, DW), jnp.float32),
        ],
    )
    conv = pl.kernel(
        _conv_body,
        out_type=jax.ShapeDtypeStruct((2, NP, H), jnp.float32),
        mesh=mesh,
        scratch_types=[
            pltpu.VMEM((ECH,), jnp.int32),
            pltpu.VMEM((ECH,), jnp.int32),
            pltpu.VMEM((NCNK, ECH), jnp.int32),
            pltpu.VMEM((ECH, H), jnp.float32),
            pltpu.VMEM((ECH, H), jnp.float32),
            pltpu.VMEM_SHARED((NP, H), jnp.float32),
            pltpu.SemaphoreType.DMA,
            pltpu.SemaphoreType.DMA,
            pltpu.SemaphoreType.DMA,
            pltpu.SemaphoreType.DMA,
        ],
    )
    return deg, conv


def _k0_body(p_ref, id_ref, dege_ref, posW_ref, posb_ref, emb_ref,
             wt_ref, wb_ref, nodeb_ref, w0_ref, n0_ref, z0_ref):
    pos = jnp.maximum(
        jnp.dot(p_ref[...], posW_ref[...], preferred_element_type=jnp.float32)
        + posb_ref[...], 0.0)
    iota = lax.broadcasted_iota(jnp.int32, (RB, NV), 1)
    oh = (iota == id_ref[...]).astype(jnp.float32)
    idf = jnp.dot(oh, jnp.maximum(emb_ref[...], 0.0),
                  preferred_element_type=jnp.float32)
    n0 = jnp.maximum(
        jnp.dot(pos, wt_ref[...], preferred_element_type=jnp.float32)
        + jnp.dot(idf, wb_ref[...], preferred_element_type=jnp.float32)
        + nodeb_ref[...], 0.0)
    dis = lax.rsqrt(dege_ref[...] + 1.0)
    z0 = jnp.dot(n0, w0_ref[...], preferred_element_type=jnp.float32) * dis
    n0_ref[...] = n0
    z0_ref[0] = z0[:, :H]
    z0_ref[1] = z0[:, H:]


def _full(shape):
    return pl.BlockSpec(shape, lambda i: tuple(0 for _ in shape))


def _k0(p, ids, deg_e, pos_W, pos_b, emb, node_W, node_b, w0):
    return pl.pallas_call(
        _k0_body,
        grid=(GRID,),
        in_specs=[
            pl.BlockSpec((RB, 3), lambda i: (i, 0)),
            pl.BlockSpec((RB, 1), lambda i: (i, 0)),
            pl.BlockSpec((RB, 1), lambda i: (i, 0)),
            _full((3, D)),
            _full((1, D)),
            _full((NV, D)),
            _full((D, D)),
            _full((D, D)),
            _full((1, D)),
            _full((D, D)),
        ],
        out_specs=[
            pl.BlockSpec((RB, D), lambda i: (i, 0)),
            pl.BlockSpec((2, RB, H), lambda i: (0, i, 0)),
        ],
        out_shape=[
            jax.ShapeDtypeStruct((NP, D), jnp.float32),
            jax.ShapeDtypeStruct((2, NP, H), jnp.float32),
        ],
    )(p, ids, deg_e, pos_W, pos_b.reshape(1, D), emb,
      node_W[:D], node_W[D:], node_b.reshape(1, D), w0)


def _stage_body(out_x, s_ref, z_ref, degc_ref, degn_ref, b_ref,
                w_ref, *out_refs):
    s = jnp.concatenate([s_ref[0], s_ref[1]], axis=1)
    z = jnp.concatenate([z_ref[0], z_ref[1]], axis=1)
    dis = lax.rsqrt(degc_ref[...] + 1.0)
    x = jnp.maximum(dis * (s + z) + b_ref[...], 0.0)
    zn = jnp.dot(x, w_ref[...], preferred_element_type=jnp.float32) \
        * lax.rsqrt(degn_ref[...] + 1.0)
    out_refs[0][0] = zn[:, :H]
    out_refs[0][1] = zn[:, H:]
    if out_x:
        out_refs[1][...] = x


def _stage(s2, z2, deg_c, deg_n, b, w, out_x):
    out_specs = [pl.BlockSpec((2, RB, H), lambda i: (0, i, 0))]
    out_shape = [jax.ShapeDtypeStruct((2, NP, H), jnp.float32)]
    if out_x:
        out_specs.append(pl.BlockSpec((RB, D), lambda i: (i, 0)))
        out_shape.append(jax.ShapeDtypeStruct((NP, D), jnp.float32))
    return pl.pallas_call(
        functools.partial(_stage_body, out_x),
        grid=(GRID,),
        in_specs=[
            pl.BlockSpec((2, RB, H), lambda i: (0, i, 0)),
            pl.BlockSpec((2, RB, H), lambda i: (0, i, 0)),
            pl.BlockSpec((RB, 1), lambda i: (i, 0)),
            pl.BlockSpec((RB, 1), lambda i: (i, 0)),
            _full((1, D)),
            _full((D, D)),
        ],
        out_specs=out_specs,
        out_shape=out_shape,
    )(s2, z2, deg_c, deg_n, b.reshape(1, D), w)


def _last_body(s_ref, z_ref, degc_ref, b_ref, x_ref):
    s = jnp.concatenate([s_ref[0], s_ref[1]], axis=1)
    z = jnp.concatenate([z_ref[0], z_ref[1]], axis=1)
    dis = lax.rsqrt(degc_ref[...] + 1.0)
    x_ref[...] = jnp.maximum(dis * (s + z) + b_ref[...], 0.0)


def _last(s2, z2, deg_c, b):
    return pl.pallas_call(
        _last_body,
        grid=(GRID,),
        in_specs=[
            pl.BlockSpec((2, RB, H), lambda i: (0, i, 0)),
            pl.BlockSpec((2, RB, H), lambda i: (0, i, 0)),
            pl.BlockSpec((RB, 1), lambda i: (i, 0)),
            _full((1, D)),
        ],
        out_specs=pl.BlockSpec((RB, D), lambda i: (i, 0)),
        out_shape=jax.ShapeDtypeStruct((NP, D), jnp.float32),
    )(s2, z2, deg_c, b.reshape(1, D))


def _pool_body(batch_ref, a_ref, b_ref, c_ref, g_ref):
    g = pl.program_id(0)
    bvals = batch_ref[...]
    start = jnp.sum((bvals < g).astype(jnp.int32))
    cnt = jnp.sum((bvals == g).astype(jnp.int32))
    c0 = start // 8
    c1 = (start + cnt + 7) // 8
    neg = jnp.full((8, D), -jnp.inf, jnp.float32)

    def _cond(carry):
        return carry[0] < c1

    def _body(carry):
        c, m0, m1, m2 = carry
        r = pl.ds(c * 8, 8)
        bm = batch_ref[r, :] == g

        def mx(acc, ref):
            return jnp.maximum(acc, jnp.where(bm, ref[r, :], -jnp.inf))

        return (c + 1, mx(m0, a_ref), mx(m1, b_ref), mx(m2, c_ref))

    _, m0, m1, m2 = lax.while_loop(_cond, _body, (c0, neg, neg, neg))
    g_ref[0, :, 0:D] = jnp.max(m0, axis=0, keepdims=True)
    g_ref[0, :, D:2 * D] = jnp.max(m1, axis=0, keepdims=True)
    g_ref[0, :, 2 * D:3 * D] = jnp.max(m2, axis=0, keepdims=True)


def _pool(batch_col, a, b, c):
    return pl.pallas_call(
        _pool_body,
        grid=(NG,),
        in_specs=[
            _full((NP, 1)),
            _full((NP, D)),
            _full((NP, D)),
            _full((NP, D)),
        ],
        out_specs=pl.BlockSpec((1, 1, 3 * D), lambda g: (g, 0, 0)),
        out_shape=jax.ShapeDtypeStruct((NG, 1, 3 * D), jnp.float32),
    )(batch_col, a, b, c).reshape(NG, 3 * D)


def _final_body(g_ref, w_ref, b_ref, out_ref):
    out_ref[...] = jnp.dot(g_ref[...], w_ref[...],
                           preferred_element_type=jnp.float32) + b_ref[...]


def _final(g, w, b):
    return pl.pallas_call(
        _final_body,
        grid=(1,),
        in_specs=[_full((NG, 3 * D)), _full((3 * D, D)), _full((1, D))],
        out_specs=_full((NG, D)),
        out_shape=jax.ShapeDtypeStruct((NG, D), jnp.float32),
    )(g, w, b.reshape(1, D))


def kernel(position_feature, id_feature, edge_index, temporal_edge_index,
           batch, pos_W, pos_b, id_emb, node_W, node_b, conv_W, conv_b,
           agg_W, agg_b):
    f32 = jnp.float32
    i32 = jnp.int32

    p = jnp.zeros((NP, 3), f32).at[:N].set(position_feature.astype(f32))
    ids = jnp.zeros((NP, 1), i32).at[:N].set(id_feature.astype(i32))
    batch_col = jnp.full((NP, 1), NG, i32).at[:N, 0].set(batch.astype(i32))

    pad_i = N + (jnp.arange(EP - E, dtype=i32) % (NP - N))
    dummy = jnp.broadcast_to(
        (N + (jnp.arange(2 * ECH, dtype=i32) % (NP - N))).reshape(1, 2, ECH),
        (16, 2, ECH))

    def prep(ei):
        s = jnp.concatenate([ei[0].astype(i32), pad_i]).reshape(16, NCNK, ECH)
        d = jnp.concatenate([ei[1].astype(i32), pad_i]).reshape(16, NCNK, ECH)
        s = jnp.concatenate([s, dummy], axis=1)
        srcs = jnp.stack([s, s + NP])
        return srcs, d

    srcs_e, dst_e = prep(edge_index)
    srcs_t, dst_t = prep(temporal_edge_index)

    _deg_kernel, _conv_kernel = _sc_kernels()
    counts = _deg_kernel(
        jnp.stack([dst_e, dst_t]).reshape(2, 16, CPT, CHUNK))
    deg_e = counts[0, :, :1]
    deg_t = counts[1, :, :1]

    n0, z0 = _k0(p, ids, deg_e, pos_W, pos_b, id_emb, node_W, node_b,
                 conv_W[0])
    s0 = _conv_kernel(z0.reshape(2 * NP, H), srcs_e, dst_e)
    (z1,) = _stage(s0, z0, deg_e, deg_t, conv_b[0], conv_W[1], out_x=False)
    s1 = _conv_kernel(z1.reshape(2 * NP, H), srcs_t, dst_t)
    z2, x2 = _stage(s1, z1, deg_t, deg_e, conv_b[1], conv_W[2], out_x=True)
    s2 = _conv_kernel(z2.reshape(2 * NP, H), srcs_e, dst_e)
    (z3,) = _stage(s2, z2, deg_e, deg_t, conv_b[2], conv_W[3], out_x=False)
    s3 = _conv_kernel(z3.reshape(2 * NP, H), srcs_t, dst_t)
    x4 = _last(s3, z3, deg_t, conv_b[3])

    g = _pool(batch_col, n0, x2, x4)
    return _final(g, agg_W, agg_b)


# speedup vs baseline: 12.7278x; 1.1255x over previous
import functools

import jax
import jax.numpy as jnp
from jax import lax
from jax.experimental import pallas as pl
from jax.experimental.pallas import tpu as pltpu
from jax.experimental.pallas import tpu_sc as plsc

N = 10000
NP = 10240
E = 160000
EP = 163840
D = 256
H = 128
NG = 64
NV = 300
CHUNK = 128
CPT = EP // 16 // CHUNK
DW = 16
ECH = 64
NCNK = EP // 16 // ECH
PAIRS = NCNK // 2
RB = 1280
GRID = NP // RB
NPT = NP // 16


def _deg_body(dst_hbm, deg_hbm, idx_v, ones_v, zbuf_v, shared):
    c = lax.axis_index("c")
    t = lax.axis_index("s")

    def _init(i, _):
        ones_v[i, :] = jnp.ones((DW,), jnp.float32)
        zbuf_v[i, :] = jnp.zeros((DW,), jnp.float32)
        return 0

    lax.fori_loop(0, CHUNK, _init, 0)
    for kk in range(NPT // CHUNK):
        pltpu.sync_copy(zbuf_v, shared.at[pl.ds(t * NPT + kk * CHUNK, CHUNK)])
    plsc.subcore_barrier()

    def _step(k, _):
        pltpu.sync_copy(dst_hbm.at[c, t, k], idx_v)
        pltpu.sync_copy(ones_v, shared.at[idx_v], add=True)
        return 0

    lax.fori_loop(0, CPT, _step, 0)
    plsc.subcore_barrier()
    pltpu.sync_copy(shared.at[pl.ds(t * NPT, NPT)],
                    deg_hbm.at[c, pl.ds(t * NPT, NPT)])


def _conv_body(z_hbm, srcs_hbm, dst_hbm, out_hbm,
               is0, is1, dslab, rows0, rows1, shared, sg0, sg1, si0, si1):
    c = lax.axis_index("c")
    t = lax.axis_index("s")
    base = t * NPT

    def _zb(i, _):
        for j in range(H // 16):
            rows0[i, pl.ds(j * 16, 16)] = jnp.zeros((16,), jnp.float32)
        return 0

    lax.fori_loop(0, ECH, _zb, 0)
    for kk in range(NPT // ECH):
        pltpu.sync_copy(rows0, shared.at[pl.ds(base + kk * ECH, ECH)])
    pltpu.sync_copy(dst_hbm.at[t], dslab)
    plsc.subcore_barrier()

    pltpu.sync_copy(srcs_hbm.at[c, t, 0], is0)
    pltpu.async_copy(z_hbm.at[is0], rows0, sg0)
    pltpu.sync_copy(srcs_hbm.at[c, t, 1], is1)
    pltpu.async_copy(z_hbm.at[is1], rows1, sg1)

    def _step(k, _):
        pltpu.make_async_copy(z_hbm.at[is0], rows0, sg0).wait()
        pltpu.async_copy(srcs_hbm.at[c, t, 2 * k + 2], is0, si0)
        pltpu.sync_copy(rows0, shared.at[dslab.at[2 * k]], add=True)
        pltpu.make_async_copy(srcs_hbm.at[c, t, 2 * k + 2], is0, si0).wait()
        pltpu.async_copy(z_hbm.at[is0], rows0, sg0)
        pltpu.make_async_copy(z_hbm.at[is1], rows1, sg1).wait()
        pltpu.async_copy(srcs_hbm.at[c, t, 2 * k + 3], is1, si1)
        pltpu.sync_copy(rows1, shared.at[dslab.at[2 * k + 1]], add=True)
        pltpu.make_async_copy(srcs_hbm.at[c, t, 2 * k + 3], is1, si1).wait()
        pltpu.async_copy(z_hbm.at[is1], rows1, sg1)
        return 0

    lax.fori_loop(0, PAIRS, _step, 0)
    pltpu.make_async_copy(z_hbm.at[is0], rows0, sg0).wait()
    pltpu.make_async_copy(z_hbm.at[is1], rows1, sg1).wait()
    plsc.subcore_barrier()
    pltpu.sync_copy(shared.at[pl.ds(base, NPT)],
                    out_hbm.at[c, pl.ds(base, NPT)])


@functools.cache
def _sc_kernels():
    mesh = plsc.VectorSubcoreMesh(core_axis_name="c", subcore_axis_name="s",
                                  num_cores=2, num_subcores=16)
    deg = pl.kernel(
        _deg_body,
        out_type=jax.ShapeDtypeStruct((2, NP, DW), jnp.float32),
        mesh=mesh,
        scratch_types=[
            pltpu.VMEM((CHUNK,), jnp.int32),
            pltpu.VMEM((CHUNK, DW), jnp.float32),
            pltpu.VMEM((CHUNK, DW), jnp.float32),
            pltpu.VMEM_SHARED((NP, DW), jnp.float32),
        ],
    )
    conv = pl.kernel(
        _conv_body,
        out_type=jax.ShapeDtypeStruct((2, NP, H), jnp.float32),
        mesh=mesh,
        scratch_types=[
            pltpu.VMEM((ECH,), jnp.int32),
            pltpu.VMEM((ECH,), jnp.int32),
            pltpu.VMEM((NCNK, ECH), jnp.int32),
            pltpu.VMEM((ECH, H), jnp.float32),
            pltpu.VMEM((ECH, H), jnp.float32),
            pltpu.VMEM_SHARED((NP, H), jnp.float32),
            pltpu.SemaphoreType.DMA,
            pltpu.SemaphoreType.DMA,
            pltpu.SemaphoreType.DMA,
            pltpu.SemaphoreType.DMA,
        ],
    )
    return deg, conv


def _k0_body(p_ref, id_ref, dege_ref, posW_ref, posb_ref, emb_ref,
             wt_ref, wb_ref, nodeb_ref, w0_ref, n0_ref, z0_ref):
    pos = jnp.maximum(
        jnp.dot(p_ref[...], posW_ref[...], preferred_element_type=jnp.float32)
        + posb_ref[...], 0.0)
    iota = lax.broadcasted_iota(jnp.int32, (RB, NV), 1)
    oh = (iota == id_ref[...]).astype(jnp.float32)
    idf = jnp.dot(oh, jnp.maximum(emb_ref[...], 0.0),
                  preferred_element_type=jnp.float32)
    n0 = jnp.maximum(
        jnp.dot(pos, wt_ref[...], preferred_element_type=jnp.float32)
        + jnp.dot(idf, wb_ref[...], preferred_element_type=jnp.float32)
        + nodeb_ref[...], 0.0)
    dis = lax.rsqrt(dege_ref[...] + 1.0)
    z0 = jnp.dot(n0, w0_ref[...], preferred_element_type=jnp.float32) * dis
    n0_ref[...] = n0
    z0_ref[0] = z0[:, :H]
    z0_ref[1] = z0[:, H:]


def _full(shape):
    return pl.BlockSpec(shape, lambda i: tuple(0 for _ in shape))


def _k0(p, ids, deg_e, pos_W, pos_b, emb, node_W, node_b, w0):
    return pl.pallas_call(
        _k0_body,
        grid=(GRID,),
        in_specs=[
            pl.BlockSpec((RB, 3), lambda i: (i, 0)),
            pl.BlockSpec((RB, 1), lambda i: (i, 0)),
            pl.BlockSpec((RB, 1), lambda i: (i, 0)),
            _full((3, D)),
            _full((1, D)),
            _full((NV, D)),
            _full((D, D)),
            _full((D, D)),
            _full((1, D)),
            _full((D, D)),
        ],
        out_specs=[
            pl.BlockSpec((RB, D), lambda i: (i, 0)),
            pl.BlockSpec((2, RB, H), lambda i: (0, i, 0)),
        ],
        out_shape=[
            jax.ShapeDtypeStruct((NP, D), jnp.float32),
            jax.ShapeDtypeStruct((2, NP, H), jnp.float32),
        ],
    )(p, ids, deg_e, pos_W, pos_b.reshape(1, D), emb,
      node_W[:D], node_W[D:], node_b.reshape(1, D), w0)


def _stage_body(out_x, s_ref, z_ref, degc_ref, degn_ref, b_ref,
                w_ref, *out_refs):
    s = jnp.concatenate([s_ref[0], s_ref[1]], axis=1)
    z = jnp.concatenate([z_ref[0], z_ref[1]], axis=1)
    dis = lax.rsqrt(degc_ref[...] + 1.0)
    x = jnp.maximum(dis * (s + z) + b_ref[...], 0.0)
    zn = jnp.dot(x, w_ref[...], preferred_element_type=jnp.float32) \
        * lax.rsqrt(degn_ref[...] + 1.0)
    out_refs[0][0] = zn[:, :H]
    out_refs[0][1] = zn[:, H:]
    if out_x:
        out_refs[1][...] = x


def _stage(s2, z2, deg_c, deg_n, b, w, out_x):
    out_specs = [pl.BlockSpec((2, RB, H), lambda i: (0, i, 0))]
    out_shape = [jax.ShapeDtypeStruct((2, NP, H), jnp.float32)]
    if out_x:
        out_specs.append(pl.BlockSpec((RB, D), lambda i: (i, 0)))
        out_shape.append(jax.ShapeDtypeStruct((NP, D), jnp.float32))
    return pl.pallas_call(
        functools.partial(_stage_body, out_x),
        grid=(GRID,),
        in_specs=[
            pl.BlockSpec((2, RB, H), lambda i: (0, i, 0)),
            pl.BlockSpec((2, RB, H), lambda i: (0, i, 0)),
            pl.BlockSpec((RB, 1), lambda i: (i, 0)),
            pl.BlockSpec((RB, 1), lambda i: (i, 0)),
            _full((1, D)),
            _full((D, D)),
        ],
        out_specs=out_specs,
        out_shape=out_shape,
    )(s2, z2, deg_c, deg_n, b.reshape(1, D), w)


def _last_body(s_ref, z_ref, degc_ref, b_ref, x_ref):
    s = jnp.concatenate([s_ref[0], s_ref[1]], axis=1)
    z = jnp.concatenate([z_ref[0], z_ref[1]], axis=1)
    dis = lax.rsqrt(degc_ref[...] + 1.0)
    x_ref[...] = jnp.maximum(dis * (s + z) + b_ref[...], 0.0)


def _last(s2, z2, deg_c, b):
    return pl.pallas_call(
        _last_body,
        grid=(GRID,),
        in_specs=[
            pl.BlockSpec((2, RB, H), lambda i: (0, i, 0)),
            pl.BlockSpec((2, RB, H), lambda i: (0, i, 0)),
            pl.BlockSpec((RB, 1), lambda i: (i, 0)),
            _full((1, D)),
        ],
        out_specs=pl.BlockSpec((RB, D), lambda i: (i, 0)),
        out_shape=jax.ShapeDtypeStruct((NP, D), jnp.float32),
    )(s2, z2, deg_c, b.reshape(1, D))


def _pool_body(batch_ref, batch2_ref, a_ref, b_ref, c_ref, g_ref):
    g = pl.program_id(0)
    b2 = batch2_ref[...]
    start = jnp.sum((b2 < g).astype(jnp.int32))
    cnt = jnp.sum((b2 == g).astype(jnp.int32))
    c0 = start // 8
    c1 = (start + cnt + 7) // 8
    neg = jnp.full((8, D), -jnp.inf, jnp.float32)

    def _cond(carry):
        return carry[0] < c1

    def _body(carry):
        c, m0, m1, m2 = carry
        r = pl.ds(c * 8, 8)
        bm = batch_ref[r, :] == g

        def mx(acc, ref):
            return jnp.maximum(acc, jnp.where(bm, ref[r, :], -jnp.inf))

        return (c + 1, mx(m0, a_ref), mx(m1, b_ref), mx(m2, c_ref))

    _, m0, m1, m2 = lax.while_loop(_cond, _body, (c0, neg, neg, neg))
    g_ref[0, :, 0:D] = jnp.max(m0, axis=0, keepdims=True)
    g_ref[0, :, D:2 * D] = jnp.max(m1, axis=0, keepdims=True)
    g_ref[0, :, 2 * D:3 * D] = jnp.max(m2, axis=0, keepdims=True)


def _pool(batch_col, batch2d, a, b, c):
    return pl.pallas_call(
        _pool_body,
        grid=(NG,),
        in_specs=[
            _full((NP, 1)),
            _full((NP // 128, 128)),
            _full((NP, D)),
            _full((NP, D)),
            _full((NP, D)),
        ],
        out_specs=pl.BlockSpec((1, 1, 3 * D), lambda g: (g, 0, 0)),
        out_shape=jax.ShapeDtypeStruct((NG, 1, 3 * D), jnp.float32),
    )(batch_col, batch2d, a, b, c).reshape(NG, 3 * D)


def _final_body(g_ref, w_ref, b_ref, out_ref):
    out_ref[...] = jnp.dot(g_ref[...], w_ref[...],
                           preferred_element_type=jnp.float32) + b_ref[...]


def _final(g, w, b):
    return pl.pallas_call(
        _final_body,
        grid=(1,),
        in_specs=[_full((NG, 3 * D)), _full((3 * D, D)), _full((1, D))],
        out_specs=_full((NG, D)),
        out_shape=jax.ShapeDtypeStruct((NG, D), jnp.float32),
    )(g, w, b.reshape(1, D))


def kernel(position_feature, id_feature, edge_index, temporal_edge_index,
           batch, pos_W, pos_b, id_emb, node_W, node_b, conv_W, conv_b,
           agg_W, agg_b):
    f32 = jnp.float32
    i32 = jnp.int32

    p = jnp.zeros((NP, 3), f32).at[:N].set(position_feature.astype(f32))
    ids = jnp.zeros((NP, 1), i32).at[:N].set(id_feature.astype(i32))
    batch_col = jnp.full((NP, 1), NG, i32).at[:N, 0].set(batch.astype(i32))

    pad_i = N + (jnp.arange(EP - E, dtype=i32) % (NP - N))
    dummy = jnp.broadcast_to(
        (N + (jnp.arange(2 * ECH, dtype=i32) % (NP - N))).reshape(1, 2, ECH),
        (16, 2, ECH))

    def prep(ei):
        s = jnp.concatenate([ei[0].astype(i32), pad_i]).reshape(16, NCNK, ECH)
        d = jnp.concatenate([ei[1].astype(i32), pad_i]).reshape(16, NCNK, ECH)
        s = jnp.concatenate([s, dummy], axis=1)
        srcs = jnp.stack([s, s + NP])
        return srcs, d

    srcs_e, dst_e = prep(edge_index)
    srcs_t, dst_t = prep(temporal_edge_index)

    _deg_kernel, _conv_kernel = _sc_kernels()
    counts = _deg_kernel(
        jnp.stack([dst_e, dst_t]).reshape(2, 16, CPT, CHUNK))
    deg_e = counts[0, :, :1]
    deg_t = counts[1, :, :1]

    n0, z0 = _k0(p, ids, deg_e, pos_W, pos_b, id_emb, node_W, node_b,
                 conv_W[0])
    s0 = _conv_kernel(z0.reshape(2 * NP, H), srcs_e, dst_e)
    (z1,) = _stage(s0, z0, deg_e, deg_t, conv_b[0], conv_W[1], out_x=False)
    s1 = _conv_kernel(z1.reshape(2 * NP, H), srcs_t, dst_t)
    z2, x2 = _stage(s1, z1, deg_t, deg_e, conv_b[1], conv_W[2], out_x=True)
    s2 = _conv_kernel(z2.reshape(2 * NP, H), srcs_e, dst_e)
    (z3,) = _stage(s2, z2, deg_e, deg_t, conv_b[2], conv_W[3], out_x=False)
    s3 = _conv_kernel(z3.reshape(2 * NP, H), srcs_t, dst_t)
    x4 = _last(s3, z3, deg_t, conv_b[3])

    g = _pool(batch_col, batch_col.reshape(NP // 128, 128), n0, x2, x4)
    return _final(g, agg_W, agg_b)
